# Initial kernel scaffold; baseline (speedup 1.0000x reference)
import jax
import jax.numpy as jnp
from jax.experimental import pallas as pl


def kernel(pair_tensor, lookup_table, embedding):
    raise NotImplementedError("write your pallas kernel here")


# speedup vs baseline: 2.2579x; 2.2579x over previous
import functools

import jax
import jax.numpy as jnp
from jax import lax
from jax.experimental import pallas as pl
from jax.experimental.pallas import tpu as pltpu
from jax.experimental.pallas import tpu_sc as plsc

_L = 16


@functools.partial(jax.jit, static_argnums=(4, 5, 6))
def _sc_lookup(p0, p1, lut, embedding, B, D, W):
    info = plsc.get_sparse_core_info()
    NW = info.num_cores * info.num_subcores
    per_w = B // NW
    CH = 512
    n_ch = per_w // CH
    mesh = plsc.VectorSubcoreMesh(core_axis_name="c", subcore_axis_name="s")

    @functools.partial(
        pl.kernel,
        mesh=mesh,
        compiler_params=pltpu.CompilerParams(use_tc_tiling_on_sc=False),
        out_type=jax.ShapeDtypeStruct((B, D), jnp.float32),
        scratch_types=[
            pltpu.VMEM((per_w,), jnp.int32),
            pltpu.VMEM((per_w,), jnp.int32),
            pltpu.VMEM((CH, D), jnp.float32),
            pltpu.VMEM((CH, D), jnp.float32),
            pltpu.SemaphoreType.DMA,
            pltpu.SemaphoreType.DMA,
        ],
    )
    def body(p0_hbm, p1_hbm, lut_hbm, emb_hbm, out_hbm,
             a_v, b_v, rows0, rows1, gsem, wsem):
        wid = lax.axis_index("s") * info.num_cores + lax.axis_index("c")
        base = wid * per_w
        pltpu.sync_copy(p0_hbm.at[pl.ds(base, per_w)], a_v)
        pltpu.sync_copy(p1_hbm.at[pl.ds(base, per_w)], b_v)

        def ix(i, carry):
            a = a_v[pl.ds(i * _L, _L)]
            b = b_v[pl.ds(i * _L, _L)]
            a_v[pl.ds(i * _L, _L)] = a * W + b
            return carry

        lax.fori_loop(0, per_w // _L, ix, 0, unroll=8)
        pltpu.async_copy(lut_hbm.at[a_v], b_v, gsem).wait()

        rows = (rows0, rows1)

        def start_gather(c, buf):
            pltpu.async_copy(
                emb_hbm.at[b_v.at[pl.ds(c * CH, CH)]], buf, gsem)

        def wait_gather(c, buf):
            pltpu.make_async_copy(
                emb_hbm.at[b_v.at[pl.ds(c * CH, CH)]], buf, gsem).wait()

        def start_write(c, buf):
            pltpu.async_copy(
                buf, out_hbm.at[pl.ds(base + c * CH, CH)], wsem)

        def drain_write(buf):
            pltpu.make_async_copy(
                buf, out_hbm.at[pl.ds(base, CH)], wsem).wait()

        def g_body(g, carry):
            for sub in range(2):
                c = g * 2 + sub
                buf = rows[sub]
                obuf = rows[1 - sub]

                @pl.when(jnp.logical_and(c >= 2, c < n_ch))
                def _():
                    drain_write(buf)

                @pl.when(c < n_ch)
                def _():
                    start_gather(c, buf)

                @pl.when(jnp.logical_and(c >= 1, c <= n_ch))
                def _():
                    wait_gather(c - 1, obuf)
                    start_write(c - 1, obuf)
            return carry

        lax.fori_loop(0, n_ch // 2 + 1, g_body, 0)
        drain_write(rows0)
        drain_write(rows1)

    return body(p0, p1, lut, embedding)


def kernel(pair_tensor, lookup_table, embedding):
    Bo, N, _ = pair_tensor.shape
    B = Bo * N
    D = embedding.shape[1]
    W = lookup_table.shape[1]
    p0 = pair_tensor[..., 0].reshape(B)
    p1 = pair_tensor[..., 1].reshape(B)
    lut = lookup_table.reshape(W * W)
    out = _sc_lookup(p0, p1, lut, embedding, B, D, W)
    return out.reshape(Bo, N, D)


# speedup vs baseline: 12.6352x; 5.5961x over previous
import functools

import jax
import jax.numpy as jnp
from jax import lax
from jax.experimental import pallas as pl
from jax.experimental.pallas import tpu as pltpu
from jax.experimental.pallas import tpu_sc as plsc

_L = 16


@functools.partial(jax.jit, static_argnums=(4, 5, 6))
def _sc_lookup(p0, p1, lut, emb_flat, B, D, W):
    info = plsc.get_sparse_core_info()
    NW = info.num_cores * info.num_subcores
    per_w = B // NW
    CH = 512
    n_ch = per_w // CH
    mesh = plsc.VectorSubcoreMesh(core_axis_name="c", subcore_axis_name="s")

    @functools.partial(
        pl.kernel,
        mesh=mesh,
        compiler_params=pltpu.CompilerParams(use_tc_tiling_on_sc=False),
        out_type=jax.ShapeDtypeStruct((B * D,), jnp.float32),
        scratch_types=[
            pltpu.VMEM((per_w,), jnp.int32),
            pltpu.VMEM((per_w,), jnp.int32),
            pltpu.VMEM((W * W,), jnp.int32),
            pltpu.SMEM((W * W,), jnp.int32),
            pltpu.VMEM((D * D,), jnp.float32),
            pltpu.VMEM((CH * D,), jnp.float32),
            pltpu.VMEM((CH * D,), jnp.float32),
            pltpu.SemaphoreType.DMA,
        ],
    )
    def body(p0_hbm, p1_hbm, lut_hbm, emb_hbm, out_hbm,
             a_v, b_v, lut_v, lut_s, emb_v, rows0, rows1, wsem):
        wid = lax.axis_index("s") * info.num_cores + lax.axis_index("c")
        base = wid * per_w
        pltpu.sync_copy(lut_hbm, lut_v)
        pltpu.sync_copy(emb_hbm, emb_v)
        pltpu.sync_copy(p0_hbm.at[pl.ds(base, per_w)], a_v)
        pltpu.sync_copy(p1_hbm.at[pl.ds(base, per_w)], b_v)

        for g in range(W * W // _L):
            lv = lut_v[pl.ds(g * _L, _L)]
            for k in range(_L):
                lut_s[g * _L + k] = lv[k]

        rows = (rows0, rows1)

        def start_write(c, buf):
            pltpu.async_copy(
                buf, out_hbm.at[pl.ds((base + c * CH) * D, CH * D)], wsem)

        def drain_write(buf):
            pltpu.make_async_copy(
                buf, out_hbm.at[pl.ds(base * D, CH * D)], wsem).wait()

        def expand_chunk(c, buf):
            r0 = c * CH

            def group(g, carry):
                av = a_v[pl.ds(r0 + g * _L, _L)]
                bv = b_v[pl.ds(r0 + g * _L, _L)]
                pos = av * W + bv
                dst0 = g * _L * D
                for i in range(_L):
                    rid = lut_s[pos[i]]
                    src = rid * D
                    dst = dst0 + i * D
                    for j in range(D // _L):
                        buf[pl.ds(dst + j * _L, _L)] = (
                            emb_v[pl.ds(src + j * _L, _L)])
                return carry

            lax.fori_loop(0, CH // _L, group, 0, unroll=2)

        def g_body(g, carry):
            for sub in range(2):
                c = g * 2 + sub
                buf = rows[sub]

                @pl.when(c >= 2)
                def _():
                    drain_write(buf)

                expand_chunk(c, buf)
                start_write(c, buf)
            return carry

        lax.fori_loop(0, n_ch // 2, g_body, 0)
        drain_write(rows0)
        drain_write(rows1)

    return body(p0, p1, lut, emb_flat)


def kernel(pair_tensor, lookup_table, embedding):
    Bo, N, _ = pair_tensor.shape
    B = Bo * N
    D = embedding.shape[1]
    W = lookup_table.shape[1]
    p0 = pair_tensor[..., 0].reshape(B)
    p1 = pair_tensor[..., 1].reshape(B)
    lut = lookup_table.reshape(W * W)
    out = _sc_lookup(p0, p1, lut, embedding.reshape(D * D), B, D, W)
    return out.reshape(Bo, N, D)


# speedup vs baseline: 17.2414x; 1.3646x over previous
import functools

import jax
import jax.numpy as jnp
from jax import lax
from jax.experimental import pallas as pl
from jax.experimental.pallas import tpu as pltpu
from jax.experimental.pallas import tpu_sc as plsc

_L = 16


@functools.partial(jax.jit, static_argnums=(4, 5, 6))
def _sc_lookup(p0, p1, lut, emb_flat, B, D, W):
    info = plsc.get_sparse_core_info()
    NW = info.num_cores * info.num_subcores
    per_w = B // NW
    CH = 512
    n_ch = per_w // CH
    mesh = plsc.VectorSubcoreMesh(core_axis_name="c", subcore_axis_name="s")

    @functools.partial(
        pl.kernel,
        mesh=mesh,
        compiler_params=pltpu.CompilerParams(
            use_tc_tiling_on_sc=False, needs_layout_passes=False),
        out_type=jax.ShapeDtypeStruct((B * D,), jnp.float32),
        scratch_types=[
            pltpu.VMEM((per_w,), jnp.int32),
            pltpu.VMEM((per_w,), jnp.int32),
            pltpu.VMEM((W * W,), jnp.int32),
            pltpu.VMEM((D * D,), jnp.float32),
            pltpu.VMEM((CH * D,), jnp.float32),
            pltpu.VMEM((CH * D,), jnp.float32),
            pltpu.SemaphoreType.DMA,
        ],
    )
    def body(p0_hbm, p1_hbm, lut_hbm, emb_hbm, out_hbm,
             a_v, b_v, lut_v, emb_v, rows0, rows1, wsem):
        wid = lax.axis_index("s") * info.num_cores + lax.axis_index("c")
        base = wid * per_w
        pltpu.sync_copy(lut_hbm, lut_v)
        pltpu.sync_copy(emb_hbm, emb_v)
        pltpu.sync_copy(p0_hbm.at[pl.ds(base, per_w)], a_v)
        pltpu.sync_copy(p1_hbm.at[pl.ds(base, per_w)], b_v)

        rows = (rows0, rows1)
        col = jnp.arange(_L, dtype=jnp.int32)

        def start_write(c, buf):
            pltpu.async_copy(
                buf, out_hbm.at[pl.ds((base + c * CH) * D, CH * D)], wsem)

        def drain_write(buf):
            pltpu.make_async_copy(
                buf, out_hbm.at[pl.ds(base * D, CH * D)], wsem).wait()

        def expand_chunk(c, buf):
            r0 = c * CH

            def group(g, carry):
                av = a_v[pl.ds(r0 + g * _L, _L)]
                bv = b_v[pl.ds(r0 + g * _L, _L)]
                pos = av * W + bv
                off = plsc.load_gather(lut_v, [pos]) * D
                dst0 = g * _L * D
                pend = []
                for i in range(_L):
                    addrs = lax.gather(
                        off, jnp.full((_L, 1), i, jnp.int32),
                        lax.GatherDimensionNumbers(
                            offset_dims=(), collapsed_slice_dims=(0,),
                            start_index_map=(0,)),
                        slice_sizes=(1,),
                        mode=lax.GatherScatterMode.PROMISE_IN_BOUNDS) + col
                    vals = [plsc.load_gather(emb_v, [addrs + j * _L])
                            for j in range(D // _L)]
                    if len(pend) == 2:
                        pdst, pvals = pend.pop(0)
                        for j, v in enumerate(pvals):
                            buf[pl.ds(pdst + j * _L, _L)] = v
                    pend.append((dst0 + i * D, vals))
                for pdst, pvals in pend:
                    for j, v in enumerate(pvals):
                        buf[pl.ds(pdst + j * _L, _L)] = v
                return carry

            lax.fori_loop(0, CH // _L, group, 0, unroll=2)

        def g_body(g, carry):
            for sub in range(2):
                c = g * 2 + sub
                buf = rows[sub]

                @pl.when(c >= 2)
                def _():
                    drain_write(buf)

                expand_chunk(c, buf)
                start_write(c, buf)
            return carry

        lax.fori_loop(0, n_ch // 2, g_body, 0)
        drain_write(rows0)
        drain_write(rows1)

    return body(p0, p1, lut, emb_flat)


def kernel(pair_tensor, lookup_table, embedding):
    Bo, N, _ = pair_tensor.shape
    B = Bo * N
    D = embedding.shape[1]
    W = lookup_table.shape[1]
    p0 = pair_tensor[..., 0].reshape(B)
    p1 = pair_tensor[..., 1].reshape(B)
    lut = lookup_table.reshape(W * W)
    out = _sc_lookup(p0, p1, lut, embedding.reshape(D * D), B, D, W)
    return out.reshape(Bo, N, D)
